# baseline (speedup 1.0000x reference)
import functools

import jax
import jax.numpy as jnp
from jax import lax
from jax.experimental import pallas as pl
from jax.experimental.pallas import tpu as pltpu
from jax.experimental.pallas import tpu_sc as plsc

_BM = 512
_CHUNK = 1024


def _argmin_body(x_ref, e_ref, x2_ref, e2_ref, codes_ref, bv_ref, bi_ref):
    c = pl.program_id(1)
    x_blk = x_ref[...]
    e_blk = e_ref[...]
    xe = lax.dot_general(x_blk, e_blk, (((1,), (1,)), ((), ())),
                         preferred_element_type=jnp.float32)
    dist = x2_ref[...] + e2_ref[...] - 2.0 * xe
    vmin = jnp.min(dist, axis=1)
    col = lax.broadcasted_iota(jnp.int32, dist.shape, 1)
    amin = jnp.min(jnp.where(dist == vmin[:, None], col, jnp.int32(2**30)),
                   axis=1) + c * _CHUNK

    @pl.when(c == 0)
    def _():
        bv_ref[...] = vmin
        bi_ref[...] = amin

    @pl.when(c != 0)
    def _():
        better = vmin < bv_ref[...]
        bv_ref[...] = jnp.where(better, vmin, bv_ref[...])
        bi_ref[...] = jnp.where(better, amin, bi_ref[...])

    @pl.when(c == (pl.num_programs(1) // 2) - 1)
    def _():
        bv_ref[...] = bv_ref[...].astype(jnp.bfloat16).astype(jnp.float32)

    @pl.when(c == pl.num_programs(1) - 1)
    def _():
        codes_ref[0, 0, :] = bi_ref[...]


def _compute_codes(x_flat, codebook):
    m, d = x_flat.shape
    n_codes = codebook.shape[0]
    grid = (m // _BM, n_codes // _CHUNK)
    x2 = jnp.sum(x_flat * x_flat, axis=1, keepdims=True)
    e2 = jnp.sum(codebook * codebook, axis=1)[None, :]
    x_rnd = x_flat.astype(jnp.bfloat16).astype(jnp.float32)
    codes3 = pl.pallas_call(
        _argmin_body,
        grid=grid,
        in_specs=[
            pl.BlockSpec((_BM, d), lambda mi, ci: (mi, 0)),
            pl.BlockSpec((_CHUNK, d), lambda mi, ci: (ci, 0)),
            pl.BlockSpec((_BM, 1), lambda mi, ci: (mi, 0)),
            pl.BlockSpec((1, _CHUNK), lambda mi, ci: (0, ci)),
        ],
        out_specs=pl.BlockSpec((1, 1, _BM), lambda mi, ci: (mi, 0, 0)),
        out_shape=jax.ShapeDtypeStruct((m // _BM, 1, _BM), jnp.int32),
        scratch_shapes=[pltpu.VMEM((_BM,), jnp.float32),
                        pltpu.VMEM((_BM,), jnp.int32)],
    )(x_rnd, codebook, x2, e2)
    return codes3.reshape(m)


_NC, _NS = 2, 16
_NW = _NC * _NS
_IDXW = 128


def _gather_rows(codebook, codes):
    m = codes.shape[0]
    d = codebook.shape[1]
    b_per_w = m // _NW
    j_chunks = b_per_w // _IDXW
    idx = codes.reshape(_NW, j_chunks, _IDXW)
    mesh = plsc.VectorSubcoreMesh(core_axis_name="c", subcore_axis_name="s")

    @functools.partial(
        pl.kernel, mesh=mesh,
        out_type=jax.ShapeDtypeStruct((_NW, j_chunks, _IDXW, d), jnp.float32),
        compiler_params=pltpu.CompilerParams(use_tc_tiling_on_sc=False),
        scratch_types=[
            pltpu.VMEM((j_chunks, _IDXW), jnp.int32),
            pltpu.VMEM((j_chunks, _IDXW, d), jnp.float32),
            pltpu.SemaphoreType.DMA,
        ],
    )
    def gk(table_hbm, idx_hbm, out_hbm, idx_v, rows_v, sem):
        wid = lax.axis_index("s") * _NC + lax.axis_index("c")
        pltpu.sync_copy(idx_hbm.at[wid], idx_v)
        cps = [pltpu.async_copy(table_hbm.at[idx_v.at[j]], rows_v.at[j], sem)
               for j in range(j_chunks)]
        for cp in cps:
            cp.wait()
        pltpu.sync_copy(rows_v, out_hbm.at[wid])

    return gk(codebook, idx).reshape(m, d)


def kernel(x, codebook):
    b, s, d = x.shape
    x_flat = x.reshape(b * s, d).astype(jnp.float32)
    e_f = codebook.astype(jnp.float32)
    codes = _compute_codes(x_flat, e_f)
    q_flat = _gather_rows(codebook, codes)
    return q_flat.reshape(b, s, d).astype(x.dtype)


# speedup vs baseline: 1.2018x; 1.2018x over previous
import functools

import jax
import jax.numpy as jnp
from jax import lax
from jax.experimental import pallas as pl
from jax.experimental.pallas import tpu as pltpu
from jax.experimental.pallas import tpu_sc as plsc

_BM = 512
_CHUNK = 1024


def _argmin_body(x_ref, e2x_ref, x2_ref, e2_ref, codes_ref,
                 av_ref, ac_ref, hv_ref, hi_ref):
    c = pl.program_id(1)
    nc = pl.num_programs(1)
    x_blk = x_ref[...]
    e_blk = e2x_ref[...]
    xe2 = lax.dot_general(x_blk, e_blk, (((1,), (1,)), ((), ())),
                          preferred_element_type=jnp.float32)
    dist = (x2_ref[...] + e2_ref[...]) - xe2

    first_of_half = (c == 0) | (c == nc // 2)

    @pl.when(first_of_half)
    def _():
        av_ref[...] = dist
        ac_ref[...] = jnp.full(dist.shape, c, jnp.int32)

    @pl.when(jnp.logical_not(first_of_half))
    def _():
        better = dist < av_ref[...]
        av_ref[...] = jnp.minimum(dist, av_ref[...])
        ac_ref[...] = jnp.where(better, jnp.int32(c), ac_ref[...])

    def _extract():
        av = av_ref[...]
        vmin = jnp.min(av, axis=1)
        col = lax.broadcasted_iota(jnp.int32, av.shape, 1)
        fidx = ac_ref[...] * _CHUNK + col
        imin = jnp.min(jnp.where(av == vmin[:, None], fidx, jnp.int32(2**30)),
                       axis=1)
        return vmin, imin

    @pl.when(c == nc // 2 - 1)
    def _():
        m0, i0 = _extract()
        hv_ref[...] = m0.astype(jnp.bfloat16).astype(jnp.float32)
        hi_ref[...] = i0

    @pl.when(c == nc - 1)
    def _():
        m1, i1 = _extract()
        better = m1 < hv_ref[...]
        codes_ref[0, 0, :] = jnp.where(better, i1, hi_ref[...])


def _compute_codes(x_flat, codebook):
    m, d = x_flat.shape
    n_codes = codebook.shape[0]
    grid = (m // _BM, n_codes // _CHUNK)
    x2 = jnp.sum(x_flat * x_flat, axis=1, keepdims=True)
    e2 = jnp.sum(codebook * codebook, axis=1)[None, :]
    x_rnd = x_flat.astype(jnp.bfloat16).astype(jnp.float32)
    codes3 = pl.pallas_call(
        _argmin_body,
        grid=grid,
        in_specs=[
            pl.BlockSpec((_BM, d), lambda mi, ci: (mi, 0)),
            pl.BlockSpec((_CHUNK, d), lambda mi, ci: (ci, 0)),
            pl.BlockSpec((_BM, 1), lambda mi, ci: (mi, 0)),
            pl.BlockSpec((1, _CHUNK), lambda mi, ci: (0, ci)),
        ],
        out_specs=pl.BlockSpec((1, 1, _BM), lambda mi, ci: (mi, 0, 0)),
        out_shape=jax.ShapeDtypeStruct((m // _BM, 1, _BM), jnp.int32),
        scratch_shapes=[pltpu.VMEM((_BM, _CHUNK), jnp.float32),
                        pltpu.VMEM((_BM, _CHUNK), jnp.int32),
                        pltpu.VMEM((_BM,), jnp.float32),
                        pltpu.VMEM((_BM,), jnp.int32)],
    )(x_rnd, codebook * 2.0, x2, e2)
    return codes3.reshape(m)


_NC, _NS = 2, 16
_NW = _NC * _NS
_IDXW = 128


def _gather_rows(codebook, codes):
    m = codes.shape[0]
    d = codebook.shape[1]
    b_per_w = m // _NW
    j_chunks = b_per_w // _IDXW
    idx = codes.reshape(_NW, j_chunks, _IDXW)
    mesh = plsc.VectorSubcoreMesh(core_axis_name="c", subcore_axis_name="s")

    @functools.partial(
        pl.kernel, mesh=mesh,
        out_type=jax.ShapeDtypeStruct((_NW, j_chunks, _IDXW, d), jnp.float32),
        compiler_params=pltpu.CompilerParams(use_tc_tiling_on_sc=False),
        scratch_types=[
            pltpu.VMEM((j_chunks, _IDXW), jnp.int32),
            pltpu.VMEM((j_chunks, _IDXW, d), jnp.float32),
            pltpu.SemaphoreType.DMA,
        ],
    )
    def gk(table_hbm, idx_hbm, out_hbm, idx_v, rows_v, sem):
        wid = lax.axis_index("s") * _NC + lax.axis_index("c")
        pltpu.sync_copy(idx_hbm.at[wid], idx_v)
        cps = [pltpu.async_copy(table_hbm.at[idx_v.at[j]], rows_v.at[j], sem)
               for j in range(j_chunks)]
        for cp in cps:
            cp.wait()
        pltpu.sync_copy(rows_v, out_hbm.at[wid])

    return gk(codebook, idx).reshape(m, d)


def kernel(x, codebook):
    b, s, d = x.shape
    x_flat = x.reshape(b * s, d).astype(jnp.float32)
    e_f = codebook.astype(jnp.float32)
    codes = _compute_codes(x_flat, e_f)
    q_flat = _gather_rows(codebook, codes)
    return q_flat.reshape(b, s, d).astype(x.dtype)


# speedup vs baseline: 1.3265x; 1.1037x over previous
import functools

import jax
import jax.numpy as jnp
from jax import lax
from jax.experimental import pallas as pl
from jax.experimental.pallas import tpu as pltpu
from jax.experimental.pallas import tpu_sc as plsc

_BM = 512
_CHUNK = 2048


def _argmin_body(x_ref, e2x_ref, x2_ref, e2_ref, codes_ref,
                 av_ref, ac_ref, hv_ref, hi_ref):
    c = pl.program_id(1)
    nc = pl.num_programs(1)
    x_blk = x_ref[...]
    e_blk = e2x_ref[...]
    xe2 = lax.dot_general(x_blk, e_blk, (((1,), (1,)), ((), ())),
                          preferred_element_type=jnp.float32)
    dist = (x2_ref[...] + e2_ref[...]) - xe2

    first_of_half = (c == 0) | (c == nc // 2)

    @pl.when(first_of_half)
    def _():
        av_ref[...] = dist
        ac_ref[...] = jnp.full(dist.shape, c, jnp.int32)

    @pl.when(jnp.logical_not(first_of_half))
    def _():
        better = dist < av_ref[...]
        av_ref[...] = jnp.minimum(dist, av_ref[...])
        ac_ref[...] = jnp.where(better, jnp.int32(c), ac_ref[...])

    def _extract():
        av = av_ref[...]
        vmin = jnp.min(av, axis=1)
        col = lax.broadcasted_iota(jnp.int32, av.shape, 1)
        fidx = ac_ref[...] * _CHUNK + col
        imin = jnp.min(jnp.where(av == vmin[:, None], fidx, jnp.int32(2**30)),
                       axis=1)
        return vmin, imin

    @pl.when(c == nc // 2 - 1)
    def _():
        m0, i0 = _extract()
        hv_ref[...] = m0.astype(jnp.bfloat16).astype(jnp.float32)
        hi_ref[...] = i0

    @pl.when(c == nc - 1)
    def _():
        m1, i1 = _extract()
        better = m1 < hv_ref[...]
        codes_ref[0, 0, :] = jnp.where(better, i1, hi_ref[...])


def _compute_codes(x_flat, codebook):
    m, d = x_flat.shape
    n_codes = codebook.shape[0]
    grid = (m // _BM, n_codes // _CHUNK)
    x2 = jnp.sum(x_flat * x_flat, axis=1, keepdims=True)
    e2 = jnp.sum(codebook * codebook, axis=1)[None, :]
    x_rnd = x_flat.astype(jnp.bfloat16).astype(jnp.float32)
    codes3 = pl.pallas_call(
        _argmin_body,
        grid=grid,
        in_specs=[
            pl.BlockSpec((_BM, d), lambda mi, ci: (mi, 0)),
            pl.BlockSpec((_CHUNK, d), lambda mi, ci: (ci, 0)),
            pl.BlockSpec((_BM, 1), lambda mi, ci: (mi, 0)),
            pl.BlockSpec((1, _CHUNK), lambda mi, ci: (0, ci)),
        ],
        out_specs=pl.BlockSpec((1, 1, _BM), lambda mi, ci: (mi, 0, 0)),
        out_shape=jax.ShapeDtypeStruct((m // _BM, 1, _BM), jnp.int32),
        scratch_shapes=[pltpu.VMEM((_BM, _CHUNK), jnp.float32),
                        pltpu.VMEM((_BM, _CHUNK), jnp.int32),
                        pltpu.VMEM((_BM,), jnp.float32),
                        pltpu.VMEM((_BM,), jnp.int32)],
    )(x_rnd, codebook * 2.0, x2, e2)
    return codes3.reshape(m)


_NC, _NS = 2, 16
_NW = _NC * _NS
_IDXW = 128


def _gather_rows(codebook, codes):
    m = codes.shape[0]
    d = codebook.shape[1]
    b_per_w = m // _NW
    j_chunks = b_per_w // _IDXW
    idx = codes.reshape(_NW, j_chunks, _IDXW)
    mesh = plsc.VectorSubcoreMesh(core_axis_name="c", subcore_axis_name="s")

    @functools.partial(
        pl.kernel, mesh=mesh,
        out_type=jax.ShapeDtypeStruct((_NW, j_chunks, _IDXW, d), jnp.float32),
        compiler_params=pltpu.CompilerParams(use_tc_tiling_on_sc=False),
        scratch_types=[
            pltpu.VMEM((j_chunks, _IDXW), jnp.int32),
            pltpu.VMEM((j_chunks, _IDXW, d), jnp.float32),
            pltpu.SemaphoreType.DMA,
        ],
    )
    def gk(table_hbm, idx_hbm, out_hbm, idx_v, rows_v, sem):
        wid = lax.axis_index("s") * _NC + lax.axis_index("c")
        pltpu.sync_copy(idx_hbm.at[wid], idx_v)
        cps = [pltpu.async_copy(table_hbm.at[idx_v.at[j]], rows_v.at[j], sem)
               for j in range(j_chunks)]
        for cp in cps:
            cp.wait()
        pltpu.sync_copy(rows_v, out_hbm.at[wid])

    return gk(codebook, idx).reshape(m, d)


def kernel(x, codebook):
    b, s, d = x.shape
    x_flat = x.reshape(b * s, d).astype(jnp.float32)
    e_f = codebook.astype(jnp.float32)
    codes = _compute_codes(x_flat, e_f)
    q_flat = _gather_rows(codebook, codes)
    return q_flat.reshape(b, s, d).astype(x.dtype)


# speedup vs baseline: 1.6381x; 1.2349x over previous
import functools

import jax
import jax.numpy as jnp
from jax import lax
from jax.experimental import pallas as pl
from jax.experimental.pallas import tpu as pltpu
from jax.experimental.pallas import tpu_sc as plsc

_BM = 512
_CHUNK = 4096


def _argmin_body(x_ref, e2x_ref, x2_ref, e2_ref, codes_ref, hv_ref, hi_ref):
    c = pl.program_id(1)
    x_blk = x_ref[...]
    e_blk = e2x_ref[...]
    xe2 = lax.dot_general(x_blk, e_blk, (((1,), (1,)), ((), ())),
                          preferred_element_type=jnp.float32)
    dist = (x2_ref[...] + e2_ref[...]) - xe2

    vmin = jnp.min(dist, axis=1)
    col = lax.broadcasted_iota(jnp.int32, dist.shape, 1)
    imin = jnp.min(jnp.where(dist == vmin[:, None], col, jnp.int32(2**30)),
                   axis=1) + c * _CHUNK

    @pl.when(c == 0)
    def _():
        hv_ref[...] = vmin.astype(jnp.bfloat16).astype(jnp.float32)
        hi_ref[...] = imin

    @pl.when(c == 1)
    def _():
        better = vmin < hv_ref[...]
        codes_ref[0, 0, :] = jnp.where(better, imin, hi_ref[...])


def _compute_codes(x_flat, codebook):
    m, d = x_flat.shape
    n_codes = codebook.shape[0]
    grid = (m // _BM, n_codes // _CHUNK)
    x2 = jnp.sum(x_flat * x_flat, axis=1, keepdims=True)
    e2 = jnp.sum(codebook * codebook, axis=1)[None, :]
    x_rnd = x_flat.astype(jnp.bfloat16).astype(jnp.float32)
    codes3 = pl.pallas_call(
        _argmin_body,
        grid=grid,
        in_specs=[
            pl.BlockSpec((_BM, d), lambda mi, ci: (mi, 0)),
            pl.BlockSpec((_CHUNK, d), lambda mi, ci: (ci, 0)),
            pl.BlockSpec((_BM, 1), lambda mi, ci: (mi, 0)),
            pl.BlockSpec((1, _CHUNK), lambda mi, ci: (0, ci)),
        ],
        out_specs=pl.BlockSpec((1, 1, _BM), lambda mi, ci: (mi, 0, 0)),
        out_shape=jax.ShapeDtypeStruct((m // _BM, 1, _BM), jnp.int32),
        scratch_shapes=[pltpu.VMEM((_BM,), jnp.float32),
                        pltpu.VMEM((_BM,), jnp.int32)],
    )(x_rnd, codebook * 2.0, x2, e2)
    return codes3.reshape(m)


_NC, _NS = 2, 16
_NW = _NC * _NS
_IDXW = 128


def _gather_rows(codebook, codes):
    m = codes.shape[0]
    d = codebook.shape[1]
    b_per_w = m // _NW
    j_chunks = b_per_w // _IDXW
    idx = codes.reshape(_NW, j_chunks, _IDXW)
    mesh = plsc.VectorSubcoreMesh(core_axis_name="c", subcore_axis_name="s")

    @functools.partial(
        pl.kernel, mesh=mesh,
        out_type=jax.ShapeDtypeStruct((_NW, j_chunks, _IDXW, d), jnp.float32),
        compiler_params=pltpu.CompilerParams(use_tc_tiling_on_sc=False),
        scratch_types=[
            pltpu.VMEM((j_chunks, _IDXW), jnp.int32),
            pltpu.VMEM((j_chunks, _IDXW, d), jnp.float32),
            pltpu.SemaphoreType.DMA,
        ],
    )
    def gk(table_hbm, idx_hbm, out_hbm, idx_v, rows_v, sem):
        wid = lax.axis_index("s") * _NC + lax.axis_index("c")
        pltpu.sync_copy(idx_hbm.at[wid], idx_v)
        cps = [pltpu.async_copy(table_hbm.at[idx_v.at[j]], rows_v.at[j], sem)
               for j in range(j_chunks)]
        for cp in cps:
            cp.wait()
        pltpu.sync_copy(rows_v, out_hbm.at[wid])

    return gk(codebook, idx).reshape(m, d)


def kernel(x, codebook):
    b, s, d = x.shape
    x_flat = x.reshape(b * s, d).astype(jnp.float32)
    e_f = codebook.astype(jnp.float32)
    codes = _compute_codes(x_flat, e_f)
    q_flat = _gather_rows(codebook, codes)
    return q_flat.reshape(b, s, d).astype(x.dtype)


# speedup vs baseline: 2.1898x; 1.3368x over previous
import functools

import jax
import jax.numpy as jnp
from jax import lax
from jax.experimental import pallas as pl
from jax.experimental.pallas import tpu as pltpu
from jax.experimental.pallas import tpu_sc as plsc

_BM = 512
_CHUNK = 4096


_SUB = 1024


def _argmin_body(x_ref, e2x_ref, x2_ref, e2_ref, codes_ref, hv_ref, hi_ref):
    c = pl.program_id(1)
    x_blk = x_ref[...]
    x2 = x2_ref[...]

    val = None
    slot = None
    for s in range(_CHUNK // _SUB):
        e_blk = e2x_ref[pl.ds(s * _SUB, _SUB), :]
        xe2 = lax.dot_general(x_blk, e_blk, (((1,), (1,)), ((), ())),
                              preferred_element_type=jnp.float32)
        d_s = (x2 + e2_ref[:, pl.ds(s * _SUB, _SUB)]) - xe2
        for j in range(_SUB // 128):
            dj = d_s[:, j * 128:(j + 1) * 128]
            g = jnp.int32(s * (_SUB // 128) + j)
            if val is None:
                val, slot = dj, jnp.full(dj.shape, g, jnp.int32)
            else:
                mask = dj < val
                val = jnp.minimum(dj, val)
                slot = jnp.where(mask, g, slot)

    vmin = jnp.min(val, axis=1)
    lane = lax.broadcasted_iota(jnp.int32, val.shape, 1)
    fidx = slot * 128 + lane
    imin = jnp.min(jnp.where(val == vmin[:, None], fidx, jnp.int32(2**30)),
                   axis=1) + c * _CHUNK

    @pl.when(c == 0)
    def _():
        hv_ref[...] = vmin.astype(jnp.bfloat16).astype(jnp.float32)
        hi_ref[...] = imin

    @pl.when(c == 1)
    def _():
        better = vmin < hv_ref[...]
        codes_ref[0, 0, :] = jnp.where(better, imin, hi_ref[...])


def _compute_codes(x_flat, codebook):
    m, d = x_flat.shape
    n_codes = codebook.shape[0]
    grid = (m // _BM, n_codes // _CHUNK)
    x2 = jnp.sum(x_flat * x_flat, axis=1, keepdims=True)
    e2 = jnp.sum(codebook * codebook, axis=1)[None, :]
    x_rnd = x_flat.astype(jnp.bfloat16).astype(jnp.float32)
    codes3 = pl.pallas_call(
        _argmin_body,
        grid=grid,
        in_specs=[
            pl.BlockSpec((_BM, d), lambda mi, ci: (mi, 0)),
            pl.BlockSpec((_CHUNK, d), lambda mi, ci: (ci, 0)),
            pl.BlockSpec((_BM, 1), lambda mi, ci: (mi, 0)),
            pl.BlockSpec((1, _CHUNK), lambda mi, ci: (0, ci)),
        ],
        out_specs=pl.BlockSpec((1, 1, _BM), lambda mi, ci: (mi, 0, 0)),
        out_shape=jax.ShapeDtypeStruct((m // _BM, 1, _BM), jnp.int32),
        scratch_shapes=[pltpu.VMEM((_BM,), jnp.float32),
                        pltpu.VMEM((_BM,), jnp.int32)],
    )(x_rnd, codebook * 2.0, x2, e2)
    return codes3.reshape(m)


_NC, _NS = 2, 16
_NW = _NC * _NS
_IDXW = 128


def _gather_rows(codebook, codes):
    m = codes.shape[0]
    d = codebook.shape[1]
    b_per_w = m // _NW
    j_chunks = b_per_w // _IDXW
    idx = codes.reshape(_NW, j_chunks, _IDXW)
    mesh = plsc.VectorSubcoreMesh(core_axis_name="c", subcore_axis_name="s")

    @functools.partial(
        pl.kernel, mesh=mesh,
        out_type=jax.ShapeDtypeStruct((_NW, j_chunks, _IDXW, d), jnp.float32),
        compiler_params=pltpu.CompilerParams(use_tc_tiling_on_sc=False),
        scratch_types=[
            pltpu.VMEM((j_chunks, _IDXW), jnp.int32),
            pltpu.VMEM((j_chunks, _IDXW, d), jnp.float32),
            pltpu.SemaphoreType.DMA,
        ],
    )
    def gk(table_hbm, idx_hbm, out_hbm, idx_v, rows_v, sem):
        wid = lax.axis_index("s") * _NC + lax.axis_index("c")
        pltpu.sync_copy(idx_hbm.at[wid], idx_v)
        cps = [pltpu.async_copy(table_hbm.at[idx_v.at[j]], rows_v.at[j], sem)
               for j in range(j_chunks)]
        for cp in cps:
            cp.wait()
        pltpu.sync_copy(rows_v, out_hbm.at[wid])

    return gk(codebook, idx).reshape(m, d)


def kernel(x, codebook):
    b, s, d = x.shape
    x_flat = x.reshape(b * s, d).astype(jnp.float32)
    e_f = codebook.astype(jnp.float32)
    codes = _compute_codes(x_flat, e_f)
    q_flat = _gather_rows(codebook, codes)
    return q_flat.reshape(b, s, d).astype(x.dtype)


# speedup vs baseline: 2.3637x; 1.0794x over previous
import functools

import jax
import jax.numpy as jnp
from jax import lax
from jax.experimental import pallas as pl
from jax.experimental.pallas import tpu as pltpu
from jax.experimental.pallas import tpu_sc as plsc

_BM = 1024
_CHUNK = 4096


_SUB = 1024


def _argmin_body(x_ref, e2x_ref, xa_ref, ea_ref, codes_ref, hv_ref, hi_ref):
    c = pl.program_id(1)
    x_blk = x_ref[...]
    xa_blk = xa_ref[...]

    val = None
    slot = None
    for s in range(_CHUNK // _SUB):
        e_blk = e2x_ref[pl.ds(s * _SUB, _SUB), :]
        xe2 = lax.dot_general(x_blk, e_blk, (((1,), (1,)), ((), ())),
                              preferred_element_type=jnp.float32)
        d_s = (xa_blk + ea_ref[:, pl.ds(s * _SUB, _SUB)]) - xe2
        for j in range(_SUB // 128):
            dj = d_s[:, j * 128:(j + 1) * 128]
            g = jnp.int32(s * (_SUB // 128) + j)
            if val is None:
                val, slot = dj, jnp.full(dj.shape, g, jnp.int32)
            else:
                mask = dj < val
                val = jnp.minimum(dj, val)
                slot = jnp.where(mask, g, slot)

    vmin = jnp.min(val, axis=1)
    lane = lax.broadcasted_iota(jnp.int32, val.shape, 1)
    fidx = slot * 128 + lane
    imin = jnp.min(jnp.where(val == vmin[:, None], fidx, jnp.int32(2**30)),
                   axis=1) + c * _CHUNK

    @pl.when(c == 0)
    def _():
        hv_ref[...] = vmin.astype(jnp.bfloat16).astype(jnp.float32)
        hi_ref[...] = imin

    @pl.when(c == 1)
    def _():
        better = vmin < hv_ref[...]
        codes_ref[0, 0, :] = jnp.where(better, imin, hi_ref[...])


def _compute_codes(x_flat, codebook):
    m, d = x_flat.shape
    n_codes = codebook.shape[0]
    grid = (m // _BM, n_codes // _CHUNK)
    x2 = jnp.sum(x_flat * x_flat, axis=1, keepdims=True)
    e2 = jnp.sum(codebook * codebook, axis=1)[None, :]
    x_rnd = x_flat.astype(jnp.bfloat16).astype(jnp.float32)
    codes3 = pl.pallas_call(
        _argmin_body,
        grid=grid,
        in_specs=[
            pl.BlockSpec((_BM, d), lambda mi, ci: (mi, 0)),
            pl.BlockSpec((_CHUNK, d), lambda mi, ci: (ci, 0)),
            pl.BlockSpec((_BM, 1), lambda mi, ci: (mi, 0)),
            pl.BlockSpec((1, _CHUNK), lambda mi, ci: (0, ci)),
        ],
        out_specs=pl.BlockSpec((1, 1, _BM), lambda mi, ci: (mi, 0, 0)),
        out_shape=jax.ShapeDtypeStruct((m // _BM, 1, _BM), jnp.int32),
        scratch_shapes=[pltpu.VMEM((_BM,), jnp.float32),
                        pltpu.VMEM((_BM,), jnp.int32)],
    )(x_rnd, codebook * 2.0, x2, e2)
    return codes3.reshape(m)


_NC, _NS = 2, 16
_NW = _NC * _NS
_IDXW = 128


def _gather_rows(codebook, codes):
    m = codes.shape[0]
    d = codebook.shape[1]
    b_per_w = m // _NW
    j_chunks = b_per_w // _IDXW
    idx = codes.reshape(_NW, j_chunks, _IDXW)
    mesh = plsc.VectorSubcoreMesh(core_axis_name="c", subcore_axis_name="s")

    @functools.partial(
        pl.kernel, mesh=mesh,
        out_type=jax.ShapeDtypeStruct((_NW, j_chunks, _IDXW, d), jnp.float32),
        compiler_params=pltpu.CompilerParams(use_tc_tiling_on_sc=False),
        scratch_types=[
            pltpu.VMEM((j_chunks, _IDXW), jnp.int32),
            pltpu.VMEM((j_chunks, _IDXW, d), jnp.float32),
            pltpu.SemaphoreType.DMA,
        ],
    )
    def gk(table_hbm, idx_hbm, out_hbm, idx_v, rows_v, sem):
        wid = lax.axis_index("s") * _NC + lax.axis_index("c")
        pltpu.sync_copy(idx_hbm.at[wid], idx_v)
        cps = [pltpu.async_copy(table_hbm.at[idx_v.at[j]], rows_v.at[j], sem)
               for j in range(j_chunks)]
        for cp in cps:
            cp.wait()
        pltpu.sync_copy(rows_v, out_hbm.at[wid])

    return gk(codebook, idx).reshape(m, d)


def kernel(x, codebook):
    b, s, d = x.shape
    x_flat = x.reshape(b * s, d).astype(jnp.float32)
    e_f = codebook.astype(jnp.float32)
    codes = _compute_codes(x_flat, e_f)
    q_flat = _gather_rows(codebook, codes)
    return q_flat.reshape(b, s, d).astype(x.dtype)


# speedup vs baseline: 2.4814x; 1.0498x over previous
import functools

import jax
import jax.numpy as jnp
from jax import lax
from jax.experimental import pallas as pl
from jax.experimental.pallas import tpu as pltpu
from jax.experimental.pallas import tpu_sc as plsc

_BM = 2048
_CHUNK = 4096


_SUB = 1024


def _argmin_body(x_ref, e2x_ref, xa_ref, ea_ref, codes_ref, hv_ref, hi_ref):
    c = pl.program_id(1)
    x_blk = x_ref[...]
    xa_blk = xa_ref[...]

    val = None
    slot = None
    for s in range(_CHUNK // _SUB):
        e_blk = e2x_ref[pl.ds(s * _SUB, _SUB), :]
        xe2 = lax.dot_general(x_blk, e_blk, (((1,), (1,)), ((), ())),
                              preferred_element_type=jnp.float32)
        d_s = (xa_blk + ea_ref[:, pl.ds(s * _SUB, _SUB)]) - xe2
        for j in range(_SUB // 128):
            dj = d_s[:, j * 128:(j + 1) * 128]
            g = jnp.int32(s * (_SUB // 128) + j)
            if val is None:
                val, slot = dj, jnp.full(dj.shape, g, jnp.int32)
            else:
                mask = dj < val
                val = jnp.minimum(dj, val)
                slot = jnp.where(mask, g, slot)

    vmin = jnp.min(val, axis=1)
    lane = lax.broadcasted_iota(jnp.int32, val.shape, 1)
    fidx = slot * 128 + lane
    imin = jnp.min(jnp.where(val == vmin[:, None], fidx, jnp.int32(2**30)),
                   axis=1) + c * _CHUNK

    @pl.when(c == 0)
    def _():
        hv_ref[...] = vmin.astype(jnp.bfloat16).astype(jnp.float32)
        hi_ref[...] = imin

    @pl.when(c == 1)
    def _():
        better = vmin < hv_ref[...]
        codes_ref[0, 0, :] = jnp.where(better, imin, hi_ref[...])


def _compute_codes(x_flat, codebook):
    m, d = x_flat.shape
    n_codes = codebook.shape[0]
    grid = (m // _BM, n_codes // _CHUNK)
    x2 = jnp.sum(x_flat * x_flat, axis=1, keepdims=True)
    e2 = jnp.sum(codebook * codebook, axis=1)[None, :]
    x_rnd = x_flat.astype(jnp.bfloat16).astype(jnp.float32)
    codes3 = pl.pallas_call(
        _argmin_body,
        grid=grid,
        in_specs=[
            pl.BlockSpec((_BM, d), lambda mi, ci: (mi, 0)),
            pl.BlockSpec((_CHUNK, d), lambda mi, ci: (ci, 0)),
            pl.BlockSpec((_BM, 1), lambda mi, ci: (mi, 0)),
            pl.BlockSpec((1, _CHUNK), lambda mi, ci: (0, ci)),
        ],
        out_specs=pl.BlockSpec((1, 1, _BM), lambda mi, ci: (mi, 0, 0)),
        out_shape=jax.ShapeDtypeStruct((m // _BM, 1, _BM), jnp.int32),
        scratch_shapes=[pltpu.VMEM((_BM,), jnp.float32),
                        pltpu.VMEM((_BM,), jnp.int32)],
    )(x_rnd, codebook * 2.0, x2, e2)
    return codes3.reshape(m)


_NC, _NS = 2, 16
_NW = _NC * _NS
_IDXW = 128


def _gather_rows(codebook, codes):
    m = codes.shape[0]
    d = codebook.shape[1]
    b_per_w = m // _NW
    j_chunks = b_per_w // _IDXW
    idx = codes.reshape(_NW, j_chunks, _IDXW)
    mesh = plsc.VectorSubcoreMesh(core_axis_name="c", subcore_axis_name="s")

    @functools.partial(
        pl.kernel, mesh=mesh,
        out_type=jax.ShapeDtypeStruct((_NW, j_chunks, _IDXW, d), jnp.float32),
        compiler_params=pltpu.CompilerParams(use_tc_tiling_on_sc=False),
        scratch_types=[
            pltpu.VMEM((j_chunks, _IDXW), jnp.int32),
            pltpu.VMEM((j_chunks, _IDXW, d), jnp.float32),
            pltpu.SemaphoreType.DMA,
        ],
    )
    def gk(table_hbm, idx_hbm, out_hbm, idx_v, rows_v, sem):
        wid = lax.axis_index("s") * _NC + lax.axis_index("c")
        pltpu.sync_copy(idx_hbm.at[wid], idx_v)
        cps = [pltpu.async_copy(table_hbm.at[idx_v.at[j]], rows_v.at[j], sem)
               for j in range(j_chunks)]
        for cp in cps:
            cp.wait()
        pltpu.sync_copy(rows_v, out_hbm.at[wid])

    return gk(codebook, idx).reshape(m, d)


def kernel(x, codebook):
    b, s, d = x.shape
    x_flat = x.reshape(b * s, d).astype(jnp.float32)
    e_f = codebook.astype(jnp.float32)
    codes = _compute_codes(x_flat, e_f)
    q_flat = _gather_rows(codebook, codes)
    return q_flat.reshape(b, s, d).astype(x.dtype)
